# baseline (speedup 1.0000x reference)
import functools

import jax
import jax.numpy as jnp
from jax import lax
from jax.experimental import pallas as pl
from jax.experimental.pallas import tpu as pltpu
from jax.experimental.pallas import tpu_sc as plsc

K = 64
BATCH = 16384
NC = 2
NS = 16
L = 16
NW = NC * NS
BPW = BATCH // NW
NG = BPW // L

_mesh = plsc.VectorSubcoreMesh(core_axis_name="c", subcore_axis_name="s")


@functools.partial(
    pl.kernel,
    out_type=jax.ShapeDtypeStruct((BATCH,), jnp.float32),
    mesh=_mesh,
    compiler_params=pltpu.CompilerParams(needs_layout_passes=False,
                                         use_tc_tiling_on_sc=False),
    scratch_types=[
        pltpu.VMEM((BPW,), jnp.int32),
        pltpu.VMEM((BPW,), jnp.int32),
        pltpu.VMEM((BPW, K), jnp.float32),
        pltpu.VMEM((BPW, K), jnp.float32),
        pltpu.VMEM((BPW,), jnp.float32),
        pltpu.VMEM((BPW,), jnp.float32),
        pltpu.VMEM((BPW,), jnp.float32),
        pltpu.SemaphoreType.DMA,
        pltpu.SemaphoreType.DMA,
        pltpu.SemaphoreType.DMA,
        pltpu.SemaphoreType.DMA,
    ],
)
def _als_sc(users_hbm, items_hbm, uf_hbm, if_hbm, ub_hbm, ib_hbm,
            out_hbm, idx_u, idx_i, u_rows, v_rows, ub_v, ib_v, out_v,
            sem_u, sem_v, sem_ub, sem_ib):
    wid = lax.axis_index("s") * NC + lax.axis_index("c")
    base = wid * BPW

    pltpu.sync_copy(users_hbm.at[pl.ds(base, BPW)], idx_u)
    pltpu.sync_copy(items_hbm.at[pl.ds(base, BPW)], idx_i)

    cu = pltpu.async_copy(uf_hbm.at[idx_u], u_rows, sem_u)
    cv = pltpu.async_copy(if_hbm.at[idx_i], v_rows, sem_v)
    cb = pltpu.async_copy(ub_hbm.at[idx_u], ub_v, sem_ub)
    ci = pltpu.async_copy(ib_hbm.at[idx_i], ib_v, sem_ib)
    cu.wait()
    cv.wait()
    cb.wait()
    ci.wait()

    iota = lax.iota(jnp.int32, L)
    for g in range(NG):
        rows = jnp.full((L,), g * L, jnp.int32) + iota
        acc0 = ub_v[pl.ds(g * L, L)] + ib_v[pl.ds(g * L, L)]

        def body(k, acc, rows=rows):
            ck = jnp.full((L,), k, jnp.int32)
            uk = plsc.load_gather(u_rows, [rows, ck])
            vk = plsc.load_gather(v_rows, [rows, ck])
            return acc + uk * vk

        out_v[pl.ds(g * L, L)] = lax.fori_loop(0, K, body, acc0)

    pltpu.sync_copy(out_v, out_hbm.at[pl.ds(base, BPW)])


def kernel(users, items, user_factors, item_factors, user_bias, item_bias):
    return _als_sc(users, items, user_factors, item_factors,
                   user_bias.reshape(-1), item_bias.reshape(-1))
